# baseline (speedup 1.0000x reference)
import functools

import jax
import jax.numpy as jnp
from jax import lax
from jax.experimental import pallas as pl
from jax.experimental.pallas import tpu as pltpu
from jax.experimental.pallas import tpu_sc as plsc

VOCAB = 1000000
T = 200
D = 32
B = 1024
N = B * T

NC, NS, L = 2, 16, 16
NW = NC * NS
PER_W = N // NW
CHUNK = 160
NCH = PER_W // CHUNK
NB = 2
LINE = 4 * D
NGRP = CHUNK // L


def _body(tok_hbm, pos_hbm, x_hbm, out_hbm,
          idx_raw, idx_line, pos_v,
          line0, line1, outb0, outb1,
          g0, g1, os0, os1):
    wid = lax.axis_index("s") * NC + lax.axis_index("c")
    base = wid * PER_W

    pltpu.sync_copy(x_hbm.at[pl.ds(base, PER_W)], idx_raw)
    pltpu.sync_copy(pos_hbm, pos_v)

    def split_idx(i, _):
        v = idx_raw[pl.ds(i * L, L)]
        idx_line[pl.ds(i * L, L)] = lax.shift_right_logical(v, 2)
        return 0
    lax.fori_loop(0, PER_W // L, split_idx, 0)

    lines = (line0, line1)
    outbs = (outb0, outb1)
    gsems = (g0, g1)
    osems = (os0, os1)

    def start_gather(c):
        return pltpu.async_copy(
            tok_hbm.at[idx_line.at[pl.ds(c * CHUNK, CHUNK)]],
            lines[c % NB], gsems[c % NB])

    def start_store(c):
        return pltpu.async_copy(
            outbs[c % 2], out_hbm.at[pl.ds(base + c * CHUNK, CHUNK)],
            osems[c % 2])

    lane = lax.iota(jnp.int32, L)

    def extract(c):
        lbuf = lines[c % NB]
        obuf = outbs[c % 2]

        tphase = (c * CHUNK) % T

        def body_g(g, _):
            rowv = g * L + lane
            offv = jnp.bitwise_and(idx_raw[pl.ds(c * CHUNK + g * L, L)], 3)
            colv = offv * D
            praw = rowv + tphase
            posrow = jnp.where(praw >= T, praw - T, praw)

            def body_w(w, _):
                sv = plsc.load_gather(lbuf, [rowv, colv + w])
                pv = plsc.load_gather(pos_v, [posrow, jnp.full((L,), 0, jnp.int32) + w])
                plsc.store_scatter(obuf, [rowv, jnp.full((L,), 0, jnp.int32) + w], sv + pv)
                return 0
            lax.fori_loop(0, D, body_w, 0)
            return 0
        lax.fori_loop(0, NGRP, body_g, 0)

    gd = [None] * NCH
    sd = [None] * NCH
    gd[0] = start_gather(0)
    for c in range(NCH):
        gd[c].wait()
        if c + 1 < NCH:
            gd[c + 1] = start_gather(c + 1)
        if c - 2 >= 0:
            sd[c - 2].wait()
        extract(c)
        sd[c] = start_store(c)
    sd[NCH - 2].wait()
    sd[NCH - 1].wait()


_mesh = plsc.VectorSubcoreMesh(core_axis_name="c", subcore_axis_name="s")

_embed = functools.partial(
    pl.kernel,
    out_type=jax.ShapeDtypeStruct((N, D), jnp.float32),
    mesh=_mesh,
    scratch_types=(
        [pltpu.VMEM((PER_W,), jnp.int32),
         pltpu.VMEM((PER_W,), jnp.int32),
         pltpu.VMEM((T, D), jnp.float32)]
        + [pltpu.VMEM((CHUNK, LINE), jnp.float32) for _ in range(NB)]
        + [pltpu.VMEM((CHUNK, D), jnp.float32) for _ in range(2)]
        + [pltpu.SemaphoreType.DMA for _ in range(NB + 2)]
    ),
    compiler_params=pltpu.CompilerParams(needs_layout_passes=False),
)(_body)


def kernel(token_table, pos_table, x):
    tok128 = token_table.reshape(VOCAB // 4, LINE)
    x_flat = x.reshape(-1).astype(jnp.int32)
    out = _embed(tok128, pos_table, x_flat)
    return out.reshape(B, T, D)


# speedup vs baseline: 1.0660x; 1.0660x over previous
import functools

import jax
import jax.numpy as jnp
from jax import lax
from jax.experimental import pallas as pl
from jax.experimental.pallas import tpu as pltpu
from jax.experimental.pallas import tpu_sc as plsc

VOCAB = 1000000
T = 200
D = 32
B = 1024
N = B * T

NC, NS, L = 2, 16, 16
NW = NC * NS
PER_W = N // NW
LINE = 4 * D
NLINES = VOCAB // 4

TG = 512
NTG = VOCAB // TG
REM = VOCAB - NTG * TG
TAIL_BASE = NTG * TG
MAX_LINE = TAIL_BASE // 4 - 1
G_PER_W = NTG // NW
SST = 33


def _tbody(tokT_hbm, tokc_hbm, slab, scr, lineb, isem, osem):
    wid = lax.axis_index("s") * NC + lax.axis_index("c")
    g_lo = wid * G_PER_W

    def do_group(c0, nvoc, nlin):
        c0 = pl.multiple_of(c0, 512)
        pltpu.sync_copy(tokT_hbm.at[:, pl.ds(c0, nvoc)], slab.at[:, pl.ds(0, nvoc)])

        def p1_d(d, _):
            def p1_c(cc, _):
                v = slab[d, pl.ds(cc * L, L)]
                plsc.store_scatter(
                    scr, [(cc * L + lax.iota(jnp.int32, L)) * SST + d], v)
                return 0
            lax.fori_loop(0, nvoc // L, p1_c, 0)
            return 0
        lax.fori_loop(0, D, p1_d, 0)

        def p2_l(l, _):
            def p2_q(q, _):
                tok = l * 4 + q // 2
                h = q % 2
                lineb[l, pl.ds(q * L, L)] = scr[pl.ds(tok * SST + h * L, L)]
                return 0
            lax.fori_loop(0, 8, p2_q, 0)
            return 0
        lax.fori_loop(0, nlin, p2_l, 0)

        pltpu.sync_copy(lineb.at[pl.ds(0, nlin)],
                        tokc_hbm.at[pl.ds(pl.multiple_of(c0 // 4, 128), nlin)])

    def body_g(k, _):
        do_group((g_lo + k) * TG, TG, TG // 4)
        return 0
    lax.fori_loop(0, G_PER_W, body_g, 0)

    @pl.when(wid == 0)
    def _():
        do_group(NTG * TG - TG, TG, TG // 4)

    return


_mesh = plsc.VectorSubcoreMesh(core_axis_name="c", subcore_axis_name="s")

_transpose = functools.partial(
    pl.kernel,
    out_type=jax.ShapeDtypeStruct((NLINES, LINE), jnp.float32),
    mesh=_mesh,
    scratch_types=[
        pltpu.VMEM((D, TG), jnp.float32),
        pltpu.VMEM((TG * SST,), jnp.float32),
        pltpu.VMEM((TG // 4, LINE), jnp.float32),
        pltpu.SemaphoreType.DMA,
        pltpu.SemaphoreType.DMA,
    ],
    compiler_params=pltpu.CompilerParams(needs_layout_passes=False),
)(_tbody)


CHUNK = 160
NCH = PER_W // CHUNK
NB = 2


def _gbody(tokc_hbm, tail_hbm, pos_hbm, x_hbm, out_hbm,
           idx_raw, idx_line, pos_v, tail_v,
           line0, line1, outb0, outb1,
           g0, g1, os0, os1):
    wid = lax.axis_index("s") * NC + lax.axis_index("c")
    base = wid * PER_W

    pltpu.sync_copy(x_hbm.at[pl.ds(base, PER_W)], idx_raw)
    pltpu.sync_copy(pos_hbm, pos_v)
    pltpu.sync_copy(tail_hbm, tail_v)

    def split_idx(i, _):
        v = idx_raw[pl.ds(i * L, L)]
        idx_line[pl.ds(i * L, L)] = jnp.minimum(
            lax.shift_right_logical(v, 2), MAX_LINE)
        return 0
    lax.fori_loop(0, PER_W // L, split_idx, 0)

    lines = (line0, line1)
    outbs = (outb0, outb1)
    gsems = (g0, g1)
    osems = (os0, os1)

    def start_gather(c):
        return pltpu.async_copy(
            tokc_hbm.at[idx_line.at[pl.ds(c * CHUNK, CHUNK)]],
            lines[c % NB], gsems[c % NB])

    def start_store(c):
        return pltpu.async_copy(
            outbs[c % 2], out_hbm.at[pl.ds(base + c * CHUNK, CHUNK)],
            osems[c % 2])

    lane = lax.iota(jnp.int32, L)

    def extract(c):
        lbuf = lines[c % NB]
        obuf = outbs[c % 2]
        tphase = (c * CHUNK) % T

        def body_j(j, _):
            grp = lax.div(j, L)
            l = lax.rem(j, L)
            rawv = idx_raw[pl.ds(c * CHUNK + grp * L, L)]
            offv = jnp.bitwise_and(rawv, 3)
            idxl = (jnp.full((L,), 0, jnp.int32) + l)[:, None]
            dn = lax.GatherDimensionNumbers(
                offset_dims=(), collapsed_slice_dims=(0,),
                start_index_map=(0,))
            off_b = lax.gather(
                offv, idxl, dimension_numbers=dn, slice_sizes=(1,),
                mode=lax.GatherScatterMode.PROMISE_IN_BOUNDS)
            raw_b = lax.gather(
                rawv, idxl, dimension_numbers=dn, slice_sizes=(1,),
                mode=lax.GatherScatterMode.PROMISE_IN_BOUNDS)
            is_tail = raw_b >= TAIL_BASE
            trow = jnp.where(is_tail, raw_b - TAIL_BASE, 0)
            praw = j + tphase
            posrow = jnp.where(praw >= T, praw - T, praw)
            rowv = jnp.full((L,), 0, jnp.int32) + j
            for h in range(2):
                colv = off_b * D + h * L + lane
                sv = plsc.load_gather(lbuf, [rowv, colv])
                tv = plsc.load_gather(tail_v, [trow, h * L + lane])
                sv = jnp.where(is_tail, tv, sv)
                pv = pos_v[posrow, pl.ds(h * L, L)]
                obuf[j, pl.ds(h * L, L)] = sv + pv
            return 0
        lax.fori_loop(0, CHUNK, body_j, 0)

    gd = [None] * NCH
    sd = [None] * NCH
    gd[0] = start_gather(0)
    for c in range(NCH):
        gd[c].wait()
        if c + 1 < NCH:
            gd[c + 1] = start_gather(c + 1)
        if c - 2 >= 0:
            sd[c - 2].wait()
        extract(c)
        sd[c] = start_store(c)
    sd[NCH - 2].wait()
    sd[NCH - 1].wait()


_embed = functools.partial(
    pl.kernel,
    out_type=jax.ShapeDtypeStruct((N, D), jnp.float32),
    mesh=_mesh,
    scratch_types=(
        [pltpu.VMEM((PER_W,), jnp.int32),
         pltpu.VMEM((PER_W,), jnp.int32),
         pltpu.VMEM((T, D), jnp.float32),
         pltpu.VMEM((REM, D), jnp.float32)]
        + [pltpu.VMEM((CHUNK, LINE), jnp.float32) for _ in range(NB)]
        + [pltpu.VMEM((CHUNK, D), jnp.float32) for _ in range(2)]
        + [pltpu.SemaphoreType.DMA for _ in range(NB + 2)]
    ),
    compiler_params=pltpu.CompilerParams(
        needs_layout_passes=False, use_tc_tiling_on_sc=False),
)(_gbody)


def kernel(token_table, pos_table, x):
    tokc = _transpose(token_table.T)
    tok_tail = token_table[TAIL_BASE:]
    x_flat = x.reshape(-1).astype(jnp.int32)
    out = _embed(tokc, tok_tail, pos_table, x_flat)
    return out.reshape(B, T, D)


# speedup vs baseline: 1.4560x; 1.3659x over previous
import functools

import jax
import jax.numpy as jnp
from jax import lax
from jax.experimental import pallas as pl
from jax.experimental.pallas import tpu as pltpu
from jax.experimental.pallas import tpu_sc as plsc

VOCAB = 1000000
T = 200
D = 32
B = 1024
N = B * T

NC, NS, L = 2, 16, 16
NW = NC * NS
PER_W = N // NW
CHUNK = 800
NCH = PER_W // CHUNK
REP = CHUNK // T
HALVES = D // L
NB = 4


def _body(tok_hbm, pos_hbm, x_hbm, out_hbm, idx_v, pos_v,
          rows0, rows1, rows2, rows3,
          g0, g1, g2, g3, s0, s1, s2, s3):
    wid = lax.axis_index("s") * NC + lax.axis_index("c")
    base = wid * PER_W

    pltpu.sync_copy(x_hbm.at[pl.ds(base, PER_W)], idx_v)
    pltpu.sync_copy(pos_hbm, pos_v)

    bufs = (rows0, rows1, rows2, rows3)
    gsems = (g0, g1, g2, g3)
    ssems = (s0, s1, s2, s3)

    def start_gather(c):
        return pltpu.async_copy(
            tok_hbm.at[idx_v.at[pl.ds(c * CHUNK, CHUNK)]],
            bufs[c % NB], gsems[c % NB])

    def start_store(c):
        return pltpu.async_copy(
            bufs[c % NB], out_hbm.at[pl.ds(base + c * CHUNK, CHUNK)],
            ssems[c % NB])

    def add_pos(buf):
        def body_t(t, _):
            for h in range(HALVES):
                pv = pos_v[t, pl.ds(h * L, L)]
                for r in range(REP):
                    j = r * T + t
                    buf[j, pl.ds(h * L, L)] = buf[j, pl.ds(h * L, L)] + pv
            return 0
        lax.fori_loop(0, T, body_t, 0)

    gd = [None] * NCH
    sd = [None] * NCH
    gd[0] = start_gather(0)
    gd[1] = start_gather(1)
    for c in range(NCH):
        gd[c].wait()
        if c + 2 < NCH:
            if c - 2 >= 0:
                sd[c - 2].wait()
            gd[c + 2] = start_gather(c + 2)
        add_pos(bufs[c % NB])
        sd[c] = start_store(c)
    for c in range(max(0, NCH - NB), NCH):
        sd[c].wait()


_mesh = plsc.VectorSubcoreMesh(core_axis_name="c", subcore_axis_name="s")

_embed = functools.partial(
    pl.kernel,
    out_type=jax.ShapeDtypeStruct((N, D), jnp.float32),
    mesh=_mesh,
    scratch_types=(
        [pltpu.VMEM((PER_W,), jnp.int32),
         pltpu.VMEM((T, D), jnp.float32)]
        + [pltpu.VMEM((CHUNK, D), jnp.float32) for _ in range(NB)]
        + [pltpu.SemaphoreType.DMA for _ in range(2 * NB)]
    ),
    compiler_params=pltpu.CompilerParams(use_tc_tiling_on_sc=False),
)(_body)


def kernel(token_table, pos_table, x):
    x_flat = x.reshape(-1).astype(jnp.int32)
    out = _embed(token_table, pos_table, x_flat)
    return out.reshape(B, T, D)
